# baseline (speedup 1.0000x reference)
import functools
import math

import jax
import jax.numpy as jnp
from jax import lax
from jax.experimental import pallas as pl
from jax.experimental.pallas import tpu as pltpu
from jax.experimental.pallas import tpu_sc as plsc

N = 100000
HIDDEN = 128
T2V_DIM = 64
DEG = 64
LANES = 16
CHUNKS = HIDDEN // LANES
GROUPS = T2V_DIM // LANES
NT = 8
RPT = DEG // NT

_SIN_COEF = (
    0.999999993788664,
    -0.16666664321701397,
    0.008333307795926153,
    -0.00019840047814565832,
    2.7527343052350527e-06,
    -2.4657468868981024e-08,
    1.3383306126652097e-10,
)
_INV_2PI = 1.0 / (2.0 * math.pi)
_2PI = 2.0 * math.pi


def _sin_poly(x):
    q = x * _INV_2PI
    half = jnp.where(q >= 0.0, 0.5, -0.5)
    rn = (q + half).astype(jnp.int32).astype(jnp.float32)
    r = x - rn * _2PI
    r2 = r * r
    p = jnp.full_like(r, _SIN_COEF[-1])
    for c in _SIN_COEF[-2::-1]:
        p = p * r2 + c
    return p * r


def _sc_body(nbr_hbm, times_hbm, t_hbm, w0_hbm, b0_hbm, ws_hbm, bs_hbm,
             Wv_hbm, bv_hbm, v_hbm, out_hbm,
             idx_v, t8_v, t1_v, w01_v, b01_v, ws_v, bs_v,
             Wv_v, bv_v, rows_v, acc_v, shared, sem):
    c = lax.axis_index("c")
    s = lax.axis_index("s")

    @pl.when(jnp.logical_and(c == 0, s < NT))
    def _():
        pltpu.sync_copy(nbr_hbm.at[pl.ds(s * RPT, RPT)], idx_v)
        gather = pltpu.async_copy(v_hbm.at[idx_v], rows_v, sem)

        pltpu.sync_copy(times_hbm.at[pl.ds(s * RPT, RPT)],
                        t8_v.at[pl.ds(0, RPT)])
        pltpu.sync_copy(t_hbm, t1_v.at[pl.ds(0, 1)])
        pltpu.sync_copy(w0_hbm, w01_v.at[pl.ds(0, 1)])
        pltpu.sync_copy(b0_hbm, b01_v.at[pl.ds(0, 1)])
        pltpu.sync_copy(ws_hbm, ws_v.at[pl.ds(0, T2V_DIM - 1)])
        pltpu.sync_copy(bs_hbm, bs_v.at[pl.ds(0, T2V_DIM - 1)])
        pltpu.sync_copy(Wv_hbm, Wv_v)
        pltpu.sync_copy(bv_hbm, bv_v)

        tval = t1_v[...][0]
        w0s = w01_v[...][0]
        b0s = b01_v[...][0]

        lane = lax.iota(jnp.int32, LANES)
        ws3 = jnp.where(lane == LANES - 1, 0.0, ws_v[pl.ds(3 * LANES, LANES)])
        bs3 = jnp.where(lane == LANES - 1, 0.0, bs_v[pl.ds(3 * LANES, LANES)])
        ws_v[pl.ds(3 * LANES, LANES)] = ws3
        bs_v[pl.ds(3 * LANES, LANES)] = bs3

        tvec = t8_v[...]
        mvec = jnp.where(jnp.logical_and(lane < RPT, tvec <= tval), 1.0, 0.0)

        cnt = mvec[0]
        st = mvec[0] * tvec[0]
        for l in range(1, RPT):
            cnt = cnt + mvec[l]
            st = st + mvec[l] * tvec[l]
        zlin = w0s * st + cnt * b0s

        wsr = [ws_v[pl.ds(g * LANES, LANES)] for g in range(GROUPS)]
        bsr = [bs_v[pl.ds(g * LANES, LANES)] for g in range(GROUPS)]
        zs = [jnp.zeros((LANES,), jnp.float32) for _ in range(GROUPS)]
        for l in range(RPT):
            tj = tvec[l]
            mj = mvec[l]
            for g in range(GROUPS):
                zs[g] = zs[g] + _sin_poly(tj * wsr[g] + bsr[g]) * mj

        accs = [
            zlin * Wv_v[0, pl.ds(k * LANES, LANES)]
            + cnt * bv_v[pl.ds(k * LANES, LANES)]
            for k in range(CHUNKS)
        ]
        for d in range(T2V_DIM - 1):
            zd = zs[d // LANES][d % LANES]
            for k in range(CHUNKS):
                accs[k] = accs[k] + zd * Wv_v[d + 1, pl.ds(k * LANES, LANES)]

        gather.wait()
        for l in range(RPT):
            m = mvec[l]
            for k in range(CHUNKS):
                accs[k] = accs[k] + rows_v[l, pl.ds(k * LANES, LANES)] * m

        for k in range(CHUNKS):
            acc_v[0, pl.ds(k * LANES, LANES)] = accs[k]
        pltpu.sync_copy(acc_v, shared.at[pl.ds(s, 1)])

    plsc.subcore_barrier()

    @pl.when(jnp.logical_and(c == 0, s == 0))
    def _():
        pltpu.sync_copy(shared, rows_v.at[pl.ds(0, NT), :])
        for k in range(CHUNKS):
            tot = rows_v[0, pl.ds(k * LANES, LANES)]
            for r in range(1, NT):
                tot = tot + rows_v[r, pl.ds(k * LANES, LANES)]
            acc_v[0, pl.ds(k * LANES, LANES)] = tot
        pltpu.sync_copy(acc_v, out_hbm)


@jax.jit
def _sc_all(v_, nbr, times, t, w0f, b0f, wsf, bsf, Wv, bv):
    mesh = plsc.VectorSubcoreMesh(core_axis_name="c", subcore_axis_name="s")
    return pl.kernel(
        _sc_body,
        out_type=jax.ShapeDtypeStruct((1, HIDDEN), jnp.float32),
        mesh=mesh,
        scratch_types=[
            pltpu.VMEM((RPT,), jnp.int32),
            pltpu.VMEM((LANES,), jnp.float32),
            pltpu.VMEM((LANES,), jnp.float32),
            pltpu.VMEM((LANES,), jnp.float32),
            pltpu.VMEM((LANES,), jnp.float32),
            pltpu.VMEM((T2V_DIM,), jnp.float32),
            pltpu.VMEM((T2V_DIM,), jnp.float32),
            pltpu.VMEM((T2V_DIM, HIDDEN), jnp.float32),
            pltpu.VMEM((HIDDEN,), jnp.float32),
            pltpu.VMEM((RPT, HIDDEN), jnp.float32),
            pltpu.VMEM((1, HIDDEN), jnp.float32),
            pltpu.VMEM_SHARED((NT, HIDDEN), jnp.float32),
            pltpu.SemaphoreType.DMA,
        ],
    )(nbr, times, t, w0f, b0f, wsf, bsf, Wv, bv, v_)


def kernel(nid, k_, q_, v_, t, neighbors, times, w0, b0, w, b,
           Wk, bk, Wq, bq, Wv, bv):
    del nid, k_, q_, Wk, bk, Wq, bq
    nbr = neighbors.astype(jnp.int32)
    return _sc_all(v_, nbr, times, t.astype(jnp.float32),
                   w0.reshape(1), b0.reshape(1),
                   w.reshape(T2V_DIM - 1), b.reshape(T2V_DIM - 1),
                   Wv, bv)


# speedup vs baseline: 1.1452x; 1.1452x over previous
import functools
import math

import jax
import jax.numpy as jnp
from jax import lax
from jax.experimental import pallas as pl
from jax.experimental.pallas import tpu as pltpu
from jax.experimental.pallas import tpu_sc as plsc

N = 100000
HIDDEN = 128
T2V_DIM = 64
DEG = 64
LANES = 16
CHUNKS = HIDDEN // LANES
GROUPS = T2V_DIM // LANES
NT = 8
RPT = DEG // NT

_SIN_COEF = (
    0.999999993788664,
    -0.16666664321701397,
    0.008333307795926153,
    -0.00019840047814565832,
    2.7527343052350527e-06,
    -2.4657468868981024e-08,
    1.3383306126652097e-10,
)
_INV_2PI = 1.0 / (2.0 * math.pi)
_2PI = 2.0 * math.pi


def _sin_poly(x):
    q = x * _INV_2PI
    half = jnp.where(q >= 0.0, 0.5, -0.5)
    rn = (q + half).astype(jnp.int32).astype(jnp.float32)
    r = x - rn * _2PI
    r2 = r * r
    p = jnp.full_like(r, _SIN_COEF[-1])
    for c in _SIN_COEF[-2::-1]:
        p = p * r2 + c
    return p * r


def _sc_body(nbr_hbm, times_hbm, t_hbm, w0_hbm, b0_hbm, ws_hbm, bs_hbm,
             Wv_hbm, bv_hbm, v_hbm, out_hbm,
             idx_v, t8_v, t1_v, w01_v, b01_v, ws_v, bs_v,
             Wv_v, bv_v, rows_v, acc_v, shared, sem, sem2):
    c = lax.axis_index("c")
    s = lax.axis_index("s")

    @pl.when(jnp.logical_and(c == 0, s < NT))
    def _():
        cps = [
            pltpu.async_copy(nbr_hbm.at[pl.ds(s * RPT, RPT)], idx_v, sem2),
            pltpu.async_copy(times_hbm.at[pl.ds(s * RPT, RPT)],
                             t8_v.at[pl.ds(0, RPT)], sem2),
            pltpu.async_copy(t_hbm, t1_v.at[pl.ds(0, 1)], sem2),
            pltpu.async_copy(w0_hbm, w01_v.at[pl.ds(0, 1)], sem2),
            pltpu.async_copy(b0_hbm, b01_v.at[pl.ds(0, 1)], sem2),
            pltpu.async_copy(ws_hbm, ws_v.at[pl.ds(0, T2V_DIM - 1)], sem2),
            pltpu.async_copy(bs_hbm, bs_v.at[pl.ds(0, T2V_DIM - 1)], sem2),
            pltpu.async_copy(Wv_hbm, Wv_v, sem2),
            pltpu.async_copy(bv_hbm, bv_v, sem2),
        ]
        cps[0].wait()
        gather = pltpu.async_copy(v_hbm.at[idx_v], rows_v, sem)
        for cp in cps[1:]:
            cp.wait()

        tval = t1_v[...][0]
        w0s = w01_v[...][0]
        b0s = b01_v[...][0]

        lane = lax.iota(jnp.int32, LANES)
        ws3 = jnp.where(lane == LANES - 1, 0.0, ws_v[pl.ds(3 * LANES, LANES)])
        bs3 = jnp.where(lane == LANES - 1, 0.0, bs_v[pl.ds(3 * LANES, LANES)])
        ws_v[pl.ds(3 * LANES, LANES)] = ws3
        bs_v[pl.ds(3 * LANES, LANES)] = bs3

        tvec = t8_v[...]
        mvec = jnp.where(jnp.logical_and(lane < RPT, tvec <= tval), 1.0, 0.0)

        cnt = mvec[0]
        st = mvec[0] * tvec[0]
        for l in range(1, RPT):
            cnt = cnt + mvec[l]
            st = st + mvec[l] * tvec[l]
        zlin = w0s * st + cnt * b0s

        wsr = [ws_v[pl.ds(g * LANES, LANES)] for g in range(GROUPS)]
        bsr = [bs_v[pl.ds(g * LANES, LANES)] for g in range(GROUPS)]
        zs = [jnp.zeros((LANES,), jnp.float32) for _ in range(GROUPS)]
        for l in range(RPT):
            tj = tvec[l]
            mj = mvec[l]
            for g in range(GROUPS):
                zs[g] = zs[g] + _sin_poly(tj * wsr[g] + bsr[g]) * mj

        accs = [
            zlin * Wv_v[0, pl.ds(k * LANES, LANES)]
            + cnt * bv_v[pl.ds(k * LANES, LANES)]
            for k in range(CHUNKS)
        ]
        for d in range(T2V_DIM - 1):
            zd = zs[d // LANES][d % LANES]
            for k in range(CHUNKS):
                accs[k] = accs[k] + zd * Wv_v[d + 1, pl.ds(k * LANES, LANES)]

        gather.wait()
        for l in range(RPT):
            m = mvec[l]
            for k in range(CHUNKS):
                accs[k] = accs[k] + rows_v[l, pl.ds(k * LANES, LANES)] * m

        for k in range(CHUNKS):
            acc_v[0, pl.ds(k * LANES, LANES)] = accs[k]
        pltpu.sync_copy(acc_v, shared.at[pl.ds(s, 1)])

    plsc.subcore_barrier()

    @pl.when(jnp.logical_and(c == 0, s == 0))
    def _():
        pltpu.sync_copy(shared, rows_v.at[pl.ds(0, NT), :])
        for k in range(CHUNKS):
            tot = rows_v[0, pl.ds(k * LANES, LANES)]
            for r in range(1, NT):
                tot = tot + rows_v[r, pl.ds(k * LANES, LANES)]
            acc_v[0, pl.ds(k * LANES, LANES)] = tot
        pltpu.sync_copy(acc_v, out_hbm)


@jax.jit
def _sc_all(v_, nbr, times, t, w0f, b0f, wsf, bsf, Wv, bv):
    mesh = plsc.VectorSubcoreMesh(core_axis_name="c", subcore_axis_name="s")
    return pl.kernel(
        _sc_body,
        out_type=jax.ShapeDtypeStruct((1, HIDDEN), jnp.float32),
        mesh=mesh,
        scratch_types=[
            pltpu.VMEM((RPT,), jnp.int32),
            pltpu.VMEM((LANES,), jnp.float32),
            pltpu.VMEM((LANES,), jnp.float32),
            pltpu.VMEM((LANES,), jnp.float32),
            pltpu.VMEM((LANES,), jnp.float32),
            pltpu.VMEM((T2V_DIM,), jnp.float32),
            pltpu.VMEM((T2V_DIM,), jnp.float32),
            pltpu.VMEM((T2V_DIM, HIDDEN), jnp.float32),
            pltpu.VMEM((HIDDEN,), jnp.float32),
            pltpu.VMEM((RPT, HIDDEN), jnp.float32),
            pltpu.VMEM((1, HIDDEN), jnp.float32),
            pltpu.VMEM_SHARED((NT, HIDDEN), jnp.float32),
            pltpu.SemaphoreType.DMA,
            pltpu.SemaphoreType.DMA,
        ],
    )(nbr, times, t, w0f, b0f, wsf, bsf, Wv, bv, v_)


def kernel(nid, k_, q_, v_, t, neighbors, times, w0, b0, w, b,
           Wk, bk, Wq, bq, Wv, bv):
    del nid, k_, q_, Wk, bk, Wq, bq
    nbr = neighbors.astype(jnp.int32)
    return _sc_all(v_, nbr, times, t.astype(jnp.float32),
                   w0.reshape(1), b0.reshape(1),
                   w.reshape(T2V_DIM - 1), b.reshape(T2V_DIM - 1),
                   Wv, bv)


# speedup vs baseline: 1.1999x; 1.0478x over previous
import functools
import math

import jax
import jax.numpy as jnp
from jax import lax
from jax.experimental import pallas as pl
from jax.experimental.pallas import tpu as pltpu
from jax.experimental.pallas import tpu_sc as plsc

N = 100000
HIDDEN = 128
T2V_DIM = 64
DEG = 64
LANES = 16
CHUNKS = HIDDEN // LANES
GROUPS = T2V_DIM // LANES
NT = 8
RPT = DEG // NT

_SIN_COEF = (
    0.999999993788664,
    -0.16666664321701397,
    0.008333307795926153,
    -0.00019840047814565832,
    2.7527343052350527e-06,
    -2.4657468868981024e-08,
    1.3383306126652097e-10,
)
_INV_2PI = 1.0 / (2.0 * math.pi)
_2PI = 2.0 * math.pi


def _sin_poly(x):
    q = x * _INV_2PI
    half = jnp.where(q >= 0.0, 0.5, -0.5)
    rn = (q + half).astype(jnp.int32).astype(jnp.float32)
    r = x - rn * _2PI
    r2 = r * r
    p = jnp.full_like(r, _SIN_COEF[-1])
    for c in _SIN_COEF[-2::-1]:
        p = p * r2 + c
    return p * r


def _sc_body(nbr_hbm, times_hbm, t_hbm, w0_hbm, b0_hbm, ws_hbm, bs_hbm,
             Wv_hbm, bv_hbm, v_hbm, out_hbm,
             idx_v, t8_v, t1_v, w01_v, b01_v, ws_v, bs_v,
             Wv_v, bv_v, rows_v, acc_v, shared, sem, sem2):
    c = lax.axis_index("c")
    s = lax.axis_index("s")

    @pl.when(jnp.logical_and(c == 0, s < NT))
    def _():
        cps = [
            pltpu.async_copy(nbr_hbm.at[pl.ds(s * RPT, RPT)], idx_v, sem2),
            pltpu.async_copy(times_hbm.at[pl.ds(s * RPT, RPT)],
                             t8_v.at[pl.ds(0, RPT)], sem2),
            pltpu.async_copy(t_hbm, t1_v.at[pl.ds(0, 1)], sem2),
            pltpu.async_copy(w0_hbm, w01_v.at[pl.ds(0, 1)], sem2),
            pltpu.async_copy(b0_hbm, b01_v.at[pl.ds(0, 1)], sem2),
            pltpu.async_copy(ws_hbm, ws_v.at[pl.ds(0, T2V_DIM - 1)], sem2),
            pltpu.async_copy(bs_hbm, bs_v.at[pl.ds(0, T2V_DIM - 1)], sem2),
            pltpu.async_copy(Wv_hbm, Wv_v, sem2),
            pltpu.async_copy(bv_hbm, bv_v, sem2),
        ]
        cps[0].wait()
        gather = pltpu.async_copy(v_hbm.at[idx_v], rows_v, sem)
        for cp in cps[1:]:
            cp.wait()

        tval = t1_v[...][0]
        w0s = w01_v[...][0]
        b0s = b01_v[...][0]

        lane = lax.iota(jnp.int32, LANES)
        ws3 = jnp.where(lane == LANES - 1, 0.0, ws_v[pl.ds(3 * LANES, LANES)])
        bs3 = jnp.where(lane == LANES - 1, 0.0, bs_v[pl.ds(3 * LANES, LANES)])
        ws_v[pl.ds(3 * LANES, LANES)] = ws3
        bs_v[pl.ds(3 * LANES, LANES)] = bs3

        tvec = t8_v[...]
        mvec = jnp.where(jnp.logical_and(lane < RPT, tvec <= tval), 1.0, 0.0)

        cnt = mvec[0]
        st = mvec[0] * tvec[0]
        for l in range(1, RPT):
            cnt = cnt + mvec[l]
            st = st + mvec[l] * tvec[l]
        zlin = w0s * st + cnt * b0s

        wsr = [ws_v[pl.ds(g * LANES, LANES)] for g in range(GROUPS)]
        bsr = [bs_v[pl.ds(g * LANES, LANES)] for g in range(GROUPS)]
        zs = [jnp.zeros((LANES,), jnp.float32) for _ in range(GROUPS)]
        for l in range(RPT):
            tj = tvec[l]
            mj = mvec[l]
            for g in range(GROUPS):
                zs[g] = zs[g] + _sin_poly(tj * wsr[g] + bsr[g]) * mj

        accs = [
            zlin * Wv_v[0, pl.ds(k * LANES, LANES)]
            + cnt * bv_v[pl.ds(k * LANES, LANES)]
            for k in range(CHUNKS)
        ]
        for d in range(T2V_DIM - 1):
            zd = zs[d // LANES][d % LANES]
            for k in range(CHUNKS):
                accs[k] = accs[k] + zd * Wv_v[d + 1, pl.ds(k * LANES, LANES)]

        gather.wait()
        for l in range(RPT):
            m = mvec[l]
            for k in range(CHUNKS):
                accs[k] = accs[k] + rows_v[l, pl.ds(k * LANES, LANES)] * m

        for k in range(CHUNKS):
            acc_v[0, pl.ds(k * LANES, LANES)] = accs[k]
        pltpu.sync_copy(acc_v, shared.at[pl.ds(s, 1)])

    plsc.subcore_barrier()

    @pl.when(jnp.logical_and(c == 0, s == 0))
    def _():
        pltpu.sync_copy(shared, rows_v.at[pl.ds(0, NT), :])
        for k in range(CHUNKS):
            tot = rows_v[0, pl.ds(k * LANES, LANES)]
            for r in range(1, NT):
                tot = tot + rows_v[r, pl.ds(k * LANES, LANES)]
            acc_v[0, pl.ds(k * LANES, LANES)] = tot
        pltpu.sync_copy(acc_v, out_hbm)


@jax.jit
def _sc_all(v_, nbr, times, t, w0f, b0f, wsf, bsf, Wv, bv):
    mesh = plsc.VectorSubcoreMesh(core_axis_name="c", subcore_axis_name="s",
                                  num_cores=1)
    return pl.kernel(
        _sc_body,
        out_type=jax.ShapeDtypeStruct((1, HIDDEN), jnp.float32),
        mesh=mesh,
        scratch_types=[
            pltpu.VMEM((RPT,), jnp.int32),
            pltpu.VMEM((LANES,), jnp.float32),
            pltpu.VMEM((LANES,), jnp.float32),
            pltpu.VMEM((LANES,), jnp.float32),
            pltpu.VMEM((LANES,), jnp.float32),
            pltpu.VMEM((T2V_DIM,), jnp.float32),
            pltpu.VMEM((T2V_DIM,), jnp.float32),
            pltpu.VMEM((T2V_DIM, HIDDEN), jnp.float32),
            pltpu.VMEM((HIDDEN,), jnp.float32),
            pltpu.VMEM((RPT, HIDDEN), jnp.float32),
            pltpu.VMEM((1, HIDDEN), jnp.float32),
            pltpu.VMEM_SHARED((NT, HIDDEN), jnp.float32),
            pltpu.SemaphoreType.DMA,
            pltpu.SemaphoreType.DMA,
        ],
    )(nbr, times, t, w0f, b0f, wsf, bsf, Wv, bv, v_)


def kernel(nid, k_, q_, v_, t, neighbors, times, w0, b0, w, b,
           Wk, bk, Wq, bq, Wv, bv):
    del nid, k_, q_, Wk, bk, Wq, bq
    nbr = neighbors.astype(jnp.int32)
    return _sc_all(v_, nbr, times, t.astype(jnp.float32),
                   w0.reshape(1), b0.reshape(1),
                   w.reshape(T2V_DIM - 1), b.reshape(T2V_DIM - 1),
                   Wv, bv)


# speedup vs baseline: 1.2184x; 1.0153x over previous
import functools
import math

import jax
import jax.numpy as jnp
from jax import lax
from jax.experimental import pallas as pl
from jax.experimental.pallas import tpu as pltpu
from jax.experimental.pallas import tpu_sc as plsc

N = 100000
HIDDEN = 128
T2V_DIM = 64
DEG = 64
LANES = 16
CHUNKS = HIDDEN // LANES
GROUPS = T2V_DIM // LANES
NT = 8
RPT = DEG // NT

_SIN_COEF = (
    0.999999993788664,
    -0.16666664321701397,
    0.008333307795926153,
    -0.00019840047814565832,
    2.7527343052350527e-06,
    -2.4657468868981024e-08,
    1.3383306126652097e-10,
)
_INV_2PI = 1.0 / (2.0 * math.pi)
_2PI = 2.0 * math.pi


def _sin_poly(x):
    q = x * _INV_2PI
    half = jnp.where(q >= 0.0, 0.5, -0.5)
    rn = (q + half).astype(jnp.int32).astype(jnp.float32)
    r = x - rn * _2PI
    r2 = r * r
    p = jnp.full_like(r, _SIN_COEF[-1])
    for c in _SIN_COEF[-2::-1]:
        p = p * r2 + c
    return p * r


def _sc_body(nbr_hbm, times_hbm, t_hbm, w0_hbm, b0_hbm, ws_hbm, bs_hbm,
             Wv_hbm, bv_hbm, v_hbm, out_hbm,
             idx_v, t8_v, t1_v, w01_v, b01_v, ws_v, bs_v,
             Wv_v, bv_v, rows_v, acc_v, shared, sem, sem2):
    c = lax.axis_index("c")
    s = lax.axis_index("s")

    @pl.when(jnp.logical_and(c == 0, s < NT))
    def _():
        cps = [
            pltpu.async_copy(nbr_hbm.at[pl.ds(s * RPT, RPT)], idx_v, sem2),
            pltpu.async_copy(times_hbm.at[pl.ds(s * RPT, RPT)],
                             t8_v.at[pl.ds(0, RPT)], sem2),
            pltpu.async_copy(t_hbm, t1_v.at[pl.ds(0, 1)], sem2),
            pltpu.async_copy(w0_hbm, w01_v.at[pl.ds(0, 1)], sem2),
            pltpu.async_copy(b0_hbm, b01_v.at[pl.ds(0, 1)], sem2),
            pltpu.async_copy(ws_hbm, ws_v.at[pl.ds(0, T2V_DIM - 1)], sem2),
            pltpu.async_copy(bs_hbm, bs_v.at[pl.ds(0, T2V_DIM - 1)], sem2),
            pltpu.async_copy(Wv_hbm, Wv_v, sem2),
            pltpu.async_copy(bv_hbm, bv_v, sem2),
        ]
        cps[0].wait()
        gather = pltpu.async_copy(v_hbm.at[idx_v], rows_v, sem)
        for cp in cps[1:7]:
            cp.wait()

        tval = t1_v[...][0]
        w0s = w01_v[...][0]
        b0s = b01_v[...][0]

        lane = lax.iota(jnp.int32, LANES)
        ws3 = jnp.where(lane == LANES - 1, 0.0, ws_v[pl.ds(3 * LANES, LANES)])
        bs3 = jnp.where(lane == LANES - 1, 0.0, bs_v[pl.ds(3 * LANES, LANES)])
        ws_v[pl.ds(3 * LANES, LANES)] = ws3
        bs_v[pl.ds(3 * LANES, LANES)] = bs3

        tvec = t8_v[...]
        mvec = jnp.where(jnp.logical_and(lane < RPT, tvec <= tval), 1.0, 0.0)

        cnt = mvec[0]
        st = mvec[0] * tvec[0]
        for l in range(1, RPT):
            cnt = cnt + mvec[l]
            st = st + mvec[l] * tvec[l]
        zlin = w0s * st + cnt * b0s

        wsr = [ws_v[pl.ds(g * LANES, LANES)] for g in range(GROUPS)]
        bsr = [bs_v[pl.ds(g * LANES, LANES)] for g in range(GROUPS)]
        zs = [jnp.zeros((LANES,), jnp.float32) for _ in range(GROUPS)]
        for l in range(RPT):
            tj = tvec[l]
            mj = mvec[l]
            for g in range(GROUPS):
                zs[g] = zs[g] + _sin_poly(tj * wsr[g] + bsr[g]) * mj

        cps[7].wait()
        cps[8].wait()

        accs = [
            zlin * Wv_v[0, pl.ds(k * LANES, LANES)]
            + cnt * bv_v[pl.ds(k * LANES, LANES)]
            for k in range(CHUNKS)
        ]
        for d in range(T2V_DIM - 1):
            zd = zs[d // LANES][d % LANES]
            for k in range(CHUNKS):
                accs[k] = accs[k] + zd * Wv_v[d + 1, pl.ds(k * LANES, LANES)]

        gather.wait()
        for l in range(RPT):
            m = mvec[l]
            for k in range(CHUNKS):
                accs[k] = accs[k] + rows_v[l, pl.ds(k * LANES, LANES)] * m

        for k in range(CHUNKS):
            acc_v[0, pl.ds(k * LANES, LANES)] = accs[k]
        pltpu.sync_copy(acc_v, shared.at[pl.ds(s, 1)])

    plsc.subcore_barrier()

    @pl.when(jnp.logical_and(c == 0, s == 0))
    def _():
        pltpu.sync_copy(shared, rows_v.at[pl.ds(0, NT), :])
        for k in range(CHUNKS):
            tot = rows_v[0, pl.ds(k * LANES, LANES)]
            for r in range(1, NT):
                tot = tot + rows_v[r, pl.ds(k * LANES, LANES)]
            acc_v[0, pl.ds(k * LANES, LANES)] = tot
        pltpu.sync_copy(acc_v, out_hbm)


@jax.jit
def _sc_all(v_, nbr, times, t, w0f, b0f, wsf, bsf, Wv, bv):
    mesh = plsc.VectorSubcoreMesh(core_axis_name="c", subcore_axis_name="s",
                                  num_cores=1)
    return pl.kernel(
        _sc_body,
        out_type=jax.ShapeDtypeStruct((1, HIDDEN), jnp.float32),
        mesh=mesh,
        scratch_types=[
            pltpu.VMEM((RPT,), jnp.int32),
            pltpu.VMEM((LANES,), jnp.float32),
            pltpu.VMEM((LANES,), jnp.float32),
            pltpu.VMEM((LANES,), jnp.float32),
            pltpu.VMEM((LANES,), jnp.float32),
            pltpu.VMEM((T2V_DIM,), jnp.float32),
            pltpu.VMEM((T2V_DIM,), jnp.float32),
            pltpu.VMEM((T2V_DIM, HIDDEN), jnp.float32),
            pltpu.VMEM((HIDDEN,), jnp.float32),
            pltpu.VMEM((RPT, HIDDEN), jnp.float32),
            pltpu.VMEM((1, HIDDEN), jnp.float32),
            pltpu.VMEM_SHARED((NT, HIDDEN), jnp.float32),
            pltpu.SemaphoreType.DMA,
            pltpu.SemaphoreType.DMA,
        ],
    )(nbr, times, t, w0f, b0f, wsf, bsf, Wv, bv, v_)


def kernel(nid, k_, q_, v_, t, neighbors, times, w0, b0, w, b,
           Wk, bk, Wq, bq, Wv, bv):
    del nid, k_, q_, Wk, bk, Wq, bq
    nbr = neighbors.astype(jnp.int32)
    return _sc_all(v_, nbr, times, t.astype(jnp.float32),
                   w0.reshape(1), b0.reshape(1),
                   w.reshape(T2V_DIM - 1), b.reshape(T2V_DIM - 1),
                   Wv, bv)
